# baseline (speedup 1.0000x reference)
import functools

import jax
import jax.numpy as jnp
from jax import lax
from jax.experimental import pallas as pl
from jax.experimental.pallas import tpu as pltpu
from jax.experimental.pallas import tpu_sc as plsc

N = 10000
E = 320000
D_IN = 128
D_OUT = 128
H = 4
D_EDGE = 16
NEG_SLOPE = 0.2

_NPAD = 10240
_NBLK = 16

_NW = 32
_G = 128
_NB = 80
_EPAD = _NW * _NB * _G
_NPT = _NPAD // 16


def _lr(v):
    return jnp.where(v >= 0, v, NEG_SLOPE * v)


def _proj_body(x_ref, w_ref, ab_ref, xp_ref, as_ref, ad_ref, bm_ref):
    x = x_ref[...]
    xp = jnp.dot(x, w_ref[...], preferred_element_type=jnp.float32)
    a256 = jnp.dot(xp, ab_ref[...], preferred_element_type=jnp.float32)
    as_ref[...] = a256[:, :128]
    ad_ref[...] = a256[:, 128:]
    bm_ref[...] = jnp.max(a256, axis=0, keepdims=True)[None]
    for h in range(H):
        xp_ref[h] = xp[:, h * D_OUT:(h + 1) * D_OUT]


def _project(xpad, W, attbig):
    blk = _NPAD // _NBLK
    return pl.pallas_call(
        _proj_body,
        grid=(_NBLK,),
        in_specs=[
            pl.BlockSpec((blk, D_IN), lambda i: (i, 0)),
            pl.BlockSpec((D_IN, H * D_OUT), lambda i: (0, 0)),
            pl.BlockSpec((H * D_OUT, 256), lambda i: (0, 0)),
        ],
        out_specs=[
            pl.BlockSpec((H, blk, D_OUT), lambda i: (0, i, 0)),
            pl.BlockSpec((blk, 128), lambda i: (i, 0)),
            pl.BlockSpec((blk, 128), lambda i: (i, 0)),
            pl.BlockSpec((1, 1, 256), lambda i: (i, 0, 0)),
        ],
        out_shape=[
            jax.ShapeDtypeStruct((H, _NPAD, D_OUT), jnp.float32),
            jax.ShapeDtypeStruct((_NPAD, 128), jnp.float32),
            jax.ShapeDtypeStruct((_NPAD, 128), jnp.float32),
            jax.ShapeDtypeStruct((_NBLK, 1, 256), jnp.float32),
        ],
    )(xpad, W, attbig)


_NQ = 320


def _att_body(src_hbm, dst_hbm, ae_hbm, as_hbm, ad_hbm, m_hbm,
              p_hbm, out_hbm,
              sv0, sv1, dv0, dv1, gs0, gs1, gd0, gd1, ab0, ab1,
              pay0, pay1, pw0, pw1, mv, acc, semm0, semm1, semg0, semg1):
    cid = lax.axis_index("c")
    sid = lax.axis_index("s")
    wid = sid * 2 + cid
    sv = (sv0, sv1)
    dv = (dv0, dv1)
    gs = (gs0, gs1)
    gd = (gd0, gd1)
    ab = (ab0, ab1)
    pay = (pay0, pay1)
    pw = (pw0, pw1)
    semm = (semm0, semm1)
    semg = (semg0, semg1)

    def _zrow(i, _):
        for v in range(8):
            pay0[i, pl.ds(v * 16, 16)] = jnp.zeros((16,), jnp.float32)
        return 0
    lax.fori_loop(0, 32, _zrow, 0)
    for z in range(_NPT // 32):
        pltpu.sync_copy(pay0, acc.at[pl.ds(sid * _NPT + z * 32, 32)])
    plsc.subcore_barrier()

    pltpu.async_copy(m_hbm, mv, semm0).wait()

    def _fire_meta(q, par):
        pltpu.async_copy(src_hbm.at[wid, q], sv[par], semm[par])
        pltpu.async_copy(dst_hbm.at[wid, q], dv[par], semm[par])
        pltpu.async_copy(ae_hbm.at[wid, q], ab[par], semm[par])

    def _drain_meta(par):
        pltpu.make_async_copy(src_hbm.at[wid, 0], sv[par], semm[par]).wait()
        pltpu.make_async_copy(dst_hbm.at[wid, 0], dv[par], semm[par]).wait()
        pltpu.make_async_copy(ae_hbm.at[wid, 0], ab[par], semm[par]).wait()

    _fire_meta(0, 0)
    _fire_meta(1, 1)

    def _pair(g, _):
        q0 = g * 2
        for par in range(2):
            _drain_meta(par)
            pltpu.async_copy(as_hbm.at[sv[par]], gs[par], semg[par])
            pltpu.async_copy(ad_hbm.at[dv[par]], gd[par], semg[par])
        for par in range(2):
            q = q0 + par
            gbase = wid * (_NQ * 32) + q * 32
            gsp, gdp, abp = gs[par], gd[par], ab[par]
            payp, pwp = pay[par], pw[par]
            pltpu.make_async_copy(as_hbm.at[pl.ds(0, 32)], gsp,
                                  semg[par]).wait()
            pltpu.make_async_copy(as_hbm.at[pl.ds(0, 32)], gdp,
                                  semg[par]).wait()

            def _row(r, _c):
                m16 = mv[pl.ds(0, 16)]
                iota = lax.iota(jnp.int32, 16)
                ae16 = abp[r, pl.ds(0, 16)]
                al = gsp[r, pl.ds(0, 16)] + gdp[r, pl.ds(0, 16)] + ae16
                al = jnp.where(al >= 0, al, NEG_SLOPE * al)
                p16 = jnp.exp(al - m16)
                vsel = jnp.where(
                    lax.broadcast_in_dim(gbase + r, (16,), ()) < E, 1.0, 0.0)
                p16 = p16 * vsel * jnp.where(iota < H, 1.0, 0.0)
                pwp[r, pl.ds(0, 16)] = p16
                aesh = ae16 * jnp.where(
                    (iota >= H) & (iota < 2 * H), 1.0, 0.0)
                dege = vsel * jnp.where(iota == 8, 1.0, 0.0)
                payp[r, pl.ds(0, 16)] = p16 + aesh + dege
                return 0
            lax.fori_loop(0, 32, _row, 0)
            pltpu.sync_copy(payp, acc.at[dv[par]], add=True)
            pltpu.sync_copy(pwp, p_hbm.at[wid, q])
            _fire_meta(q0 + 2 + par, par)
        return 0

    lax.fori_loop(0, _NQ // 2, _pair, 0)
    _drain_meta(0)
    _drain_meta(1)

    plsc.subcore_barrier()
    pltpu.sync_copy(acc.at[pl.ds(sid * _NPT, _NPT)],
                    out_hbm.at[cid, pl.ds(sid * _NPT, _NPT)])


def _sc_attention(src2, dst2, ae16r, a16s, a16d, m16):
    mesh = plsc.VectorSubcoreMesh(core_axis_name="c", subcore_axis_name="s")
    kern = pl.kernel(
        _att_body,
        mesh=mesh,
        out_type=[
            jax.ShapeDtypeStruct((_NW, _NQ + 2, 32, 16), jnp.float32),
            jax.ShapeDtypeStruct((2, _NPAD, D_OUT), jnp.float32),
        ],
        scratch_types=(
            [pltpu.VMEM((32,), jnp.int32)] * 4
            + [pltpu.VMEM((32, 128), jnp.float32)] * 4
            + [pltpu.VMEM((32, 16), jnp.float32)] * 2
            + [pltpu.VMEM((32, D_OUT), jnp.float32)] * 2
            + [pltpu.VMEM((32, 16), jnp.float32)] * 2
            + [pltpu.VMEM((16,), jnp.float32)]
            + [pltpu.VMEM_SHARED((_NPAD, D_OUT), jnp.float32)]
            + [pltpu.SemaphoreType.DMA] * 4
        ),
    )
    return kern(src2, dst2, ae16r, a16s, a16d, m16)


_GM = 32
_NBM = 320
_NPADM = 10112
_NPTM = _NPADM // 16


def _msg_body(meta_hbm, dst_hbm, p_hbm, winv_hbm, xp4_hbm, out_hbm,
              meta0, meta1, dv0, dv1, pc0, pc1,
              gv0, gv1, ra0, rb0, rc0, rd0, ra1, rb1, rc1, rd1,
              acc, semm0, semm1, semg0, semg1):
    cid = lax.axis_index("c")
    sid = lax.axis_index("s")
    wid = sid * 2 + cid
    meta = (meta0, meta1)
    dv = (dv0, dv1)
    pc = (pc0, pc1)
    gv = (gv0, gv1)
    rows = ((ra0, rb0, rc0, rd0), (ra1, rb1, rc1, rd1))
    semm = (semm0, semm1)
    semg = (semg0, semg1)

    def _zrow(i, _):
        for v in range(8):
            rd0[i, pl.ds(v * 16, 16)] = jnp.zeros((16,), jnp.float32)
        return 0
    lax.fori_loop(0, _GM, _zrow, 0)
    for z in range(19):
        pltpu.sync_copy(rd0, acc.at[pl.ds(sid * _NPTM + z * _GM, _GM)])
    pltpu.sync_copy(rd0.at[pl.ds(0, 24)],
                    acc.at[pl.ds(sid * _NPTM + 608, 24)])
    plsc.subcore_barrier()

    def _fire_meta(b, par):
        pltpu.async_copy(meta_hbm.at[wid, b], meta[par], semm[par])
        pltpu.async_copy(dst_hbm.at[wid, b], dv[par], semm[par])
        pltpu.async_copy(p_hbm.at[wid, b], pc[par], semm[par])

    def _drain_meta(par):
        pltpu.make_async_copy(meta_hbm.at[wid, 0], meta[par],
                              semm[par]).wait()
        pltpu.make_async_copy(dst_hbm.at[wid, 0], dv[par], semm[par]).wait()
        pltpu.make_async_copy(p_hbm.at[wid, 0], pc[par], semm[par]).wait()

    def _fire_gathers(par):
        for h in range(H):
            pltpu.async_copy(
                xp4_hbm.at[h].at[meta[par].at[0, pl.ds(0, _GM)]],
                rows[par][h], semg[par])
        pltpu.async_copy(winv_hbm.at[dv[par]], gv[par], semg[par])

    def _drain_gathers(par):
        for h in range(H):
            pltpu.make_async_copy(winv_hbm.at[pl.ds(0, _GM)],
                                  rows[par][h], semg[par]).wait()
        pltpu.make_async_copy(winv_hbm.at[pl.ds(0, _GM)], gv[par],
                              semg[par]).wait()

    _fire_meta(0, 0)
    _fire_meta(1, 1)

    def _pair(g, _):
        b0 = g * 2
        for par in range(2):
            _drain_meta(par)
            _fire_gathers(par)
        for par in range(2):
            r0p, r1p, r2p, r3p = rows[par]
            msg = r3p
            pcp = pc[par]
            gvp = gv[par]
            _drain_gathers(par)

            def _row(r, _c):
                wrow = (pcp[r // 8, pl.ds((r % 8) * 16, 16)]
                        * gvp[r, pl.ds(0, 16)])
                w0 = lax.broadcast_in_dim(wrow[0], (16,), ())
                w1 = lax.broadcast_in_dim(wrow[1], (16,), ())
                w2 = lax.broadcast_in_dim(wrow[2], (16,), ())
                w3 = lax.broadcast_in_dim(wrow[3], (16,), ())
                for v in range(8):
                    sl = pl.ds(v * 16, 16)
                    a = r0p[r, sl] * w0 + r1p[r, sl] * w1
                    a = a + r2p[r, sl] * w2 + r3p[r, sl] * w3
                    msg[r, sl] = a
                return 0
            lax.fori_loop(0, _GM, _row, 0)
            pltpu.sync_copy(msg, acc.at[dv[par]], add=True)
            _fire_meta(b0 + 2 + par, par)
        return 0

    lax.fori_loop(0, _NBM // 2, _pair, 0)
    _drain_meta(0)
    _drain_meta(1)

    plsc.subcore_barrier()
    pltpu.sync_copy(acc.at[pl.ds(sid * _NPTM, _NPTM)],
                    out_hbm.at[cid, pl.ds(sid * _NPTM, _NPTM)])


def _sc_message(meta, dst3, p3, winvtbl, xp4):
    mesh = plsc.VectorSubcoreMesh(core_axis_name="c", subcore_axis_name="s")
    kern = pl.kernel(
        _msg_body,
        mesh=mesh,
        out_type=jax.ShapeDtypeStruct((2, _NPADM, D_OUT), jnp.float32),
        scratch_types=(
            [pltpu.VMEM((1, 128), jnp.int32)] * 2
            + [pltpu.VMEM((_GM,), jnp.int32)] * 2
            + [pltpu.VMEM((4, 128), jnp.float32)] * 2
            + [pltpu.VMEM((_GM, D_OUT), jnp.float32)] * 2
            + [pltpu.VMEM((_GM, D_OUT), jnp.float32)] * 8
            + [pltpu.VMEM_SHARED((_NPADM, D_OUT), jnp.float32)]
            + [pltpu.SemaphoreType.DMA] * 4
        ),
    )
    return kern(meta, dst3, p3, winvtbl, xp4)


def kernel(x, edge_index, edge_attr, W, att_src, att_dst, W_e, att_edge):
    src, dst = edge_index[0], edge_index[1]
    B = jnp.einsum('khd,hd->kh', W_e.reshape(D_EDGE, H, D_OUT), att_edge)

    attbig = jnp.zeros((H * D_OUT, 256), jnp.float32)
    attbig = attbig.at[:, :H].set(
        jax.scipy.linalg.block_diag(*[att_src[h][:, None] for h in range(H)]))
    attbig = attbig.at[:, 128:128 + H].set(
        jax.scipy.linalg.block_diag(*[att_dst[h][:, None] for h in range(H)]))
    xpad = jnp.zeros((_NPAD, D_IN), jnp.float32).at[:N].set(x)
    xp4, a16s, a16d, bm = _project(xpad, W, attbig)
    asrc = a16s[:N, :H]
    adst = a16d[:N, :H]
    msd = jnp.max(bm.reshape(_NBLK, 256).T, axis=1)

    ae = edge_attr @ B
    ae16r = jnp.zeros((_EPAD, 16), jnp.float32)
    ae16r = ae16r.at[:E, :H].set(ae).at[:E, H:2 * H].set(ae)
    ae16r = jnp.zeros((_NW, _NQ + 2, 32, 16), jnp.float32).at[:, :_NQ].set(
        ae16r.reshape(_NW, _NQ, 32, 16))
    aemax = jnp.max(ae.T, axis=1)
    M = _lr(msd[:H] + msd[128:128 + H] + jnp.maximum(aemax, 0.0))
    m16 = jnp.zeros((16,), jnp.float32).at[:H].set(M)

    srcp = jnp.zeros((_EPAD,), jnp.int32).at[:E].set(src)
    dstp = jnp.zeros((_EPAD,), jnp.int32).at[:E].set(dst)
    src2 = jnp.zeros((_NW, _NQ + 2, 32), jnp.int32).at[:, :_NQ].set(
        srcp.reshape(_NW, _NQ, 32))
    dst2 = jnp.zeros((_NW, _NQ + 2, 32), jnp.int32).at[:, :_NQ].set(
        dstp.reshape(_NW, _NQ, 32))
    pr, sparts = _sc_attention(src2, dst2, ae16r, a16s, a16d, m16)
    pr = pr[:, :_NQ]

    sacc = sparts[0, :N] + sparts[1, :N]
    deg = sacc[:, 8]
    sA = sacc[:, 4:4 + H]
    sE = sacc[:, :H]
    a_loop = _lr(asrc + adst + sA / jnp.maximum(deg, 1.0)[:, None])
    p_loop = jnp.exp(a_loop - M[None, :])
    s = sE + p_loop
    winv = 1.0 / s
    w_loop = p_loop * winv

    winvtbl = jnp.zeros((_NPAD, 128), jnp.float32).at[:N, :H].set(winv)
    srcrows = jnp.zeros((_NW, _NBM + 2, 128), jnp.int32).at[:, :_NBM, :_GM].set(
        srcp.reshape(_NW, _NBM, _GM))
    meta = srcrows.reshape(_NW, _NBM + 2, 1, 128)
    p3 = jnp.zeros((_NW, _NBM + 2, 4, 128), jnp.float32).at[:, :_NBM].set(
        pr.reshape(_NW, _NBM, 4, 128))
    dst3 = jnp.zeros((_NW, _NBM + 2, _GM), jnp.int32).at[:, :_NBM].set(
        dstp.reshape(_NW, _NBM, _GM))
    parts = _sc_message(meta, dst3, p3, winvtbl, xp4)

    self_msg = jnp.zeros((N, D_OUT), jnp.float32)
    for h in range(H):
        self_msg = self_msg + w_loop[:, h:h + 1] * xp4[h, :N]
    out = (parts[0, :N] + parts[1, :N] + self_msg) * (1.0 / H)
    return out


# speedup vs baseline: 1.0485x; 1.0485x over previous
import functools

import jax
import jax.numpy as jnp
from jax import lax
from jax.experimental import pallas as pl
from jax.experimental.pallas import tpu as pltpu
from jax.experimental.pallas import tpu_sc as plsc

N = 10000
E = 320000
D_IN = 128
D_OUT = 128
H = 4
D_EDGE = 16
NEG_SLOPE = 0.2

_NPAD = 10240
_NBLK = 16

_NW = 32
_G = 128
_NB = 80
_EPAD = _NW * _NB * _G
_NPT = _NPAD // 16


def _lr(v):
    return jnp.where(v >= 0, v, NEG_SLOPE * v)


def _proj_body(x_ref, w_ref, ab_ref, xp_ref, as_ref, ad_ref, bm_ref):
    x = x_ref[...]
    xp = jnp.dot(x, w_ref[...], preferred_element_type=jnp.float32)
    a256 = jnp.dot(xp, ab_ref[...], preferred_element_type=jnp.float32)
    as_ref[...] = a256[:, :128]
    ad_ref[...] = a256[:, 128:]
    bm_ref[...] = jnp.max(a256, axis=0, keepdims=True)[None]
    for h in range(H):
        xp_ref[h] = xp[:, h * D_OUT:(h + 1) * D_OUT]


def _project(xpad, W, attbig):
    blk = _NPAD // _NBLK
    return pl.pallas_call(
        _proj_body,
        grid=(_NBLK,),
        in_specs=[
            pl.BlockSpec((blk, D_IN), lambda i: (i, 0)),
            pl.BlockSpec((D_IN, H * D_OUT), lambda i: (0, 0)),
            pl.BlockSpec((H * D_OUT, 256), lambda i: (0, 0)),
        ],
        out_specs=[
            pl.BlockSpec((H, blk, D_OUT), lambda i: (0, i, 0)),
            pl.BlockSpec((blk, 128), lambda i: (i, 0)),
            pl.BlockSpec((blk, 128), lambda i: (i, 0)),
            pl.BlockSpec((1, 1, 256), lambda i: (i, 0, 0)),
        ],
        out_shape=[
            jax.ShapeDtypeStruct((H, _NPAD, D_OUT), jnp.float32),
            jax.ShapeDtypeStruct((_NPAD, 128), jnp.float32),
            jax.ShapeDtypeStruct((_NPAD, 128), jnp.float32),
            jax.ShapeDtypeStruct((_NBLK, 1, 256), jnp.float32),
        ],
    )(xpad, W, attbig)


_NQ = 320


def _att_body(src_hbm, dst_hbm, ae_hbm, as_hbm, ad_hbm, m_hbm,
              p_hbm, out_hbm,
              sv0, sv1, dv0, dv1, gs0, gs1, gd0, gd1, ab0, ab1,
              pay0, pay1, pw0, pw1, mv, acc, semm0, semm1, semg0, semg1):
    cid = lax.axis_index("c")
    sid = lax.axis_index("s")
    wid = sid * 2 + cid
    sv = (sv0, sv1)
    dv = (dv0, dv1)
    gs = (gs0, gs1)
    gd = (gd0, gd1)
    ab = (ab0, ab1)
    pay = (pay0, pay1)
    pw = (pw0, pw1)
    semm = (semm0, semm1)
    semg = (semg0, semg1)

    def _zrow(i, _):
        for v in range(8):
            pay0[i, pl.ds(v * 16, 16)] = jnp.zeros((16,), jnp.float32)
        return 0
    lax.fori_loop(0, 32, _zrow, 0)
    for z in range(_NPT // 32):
        pltpu.sync_copy(pay0, acc.at[pl.ds(sid * _NPT + z * 32, 32)])
    plsc.subcore_barrier()

    pltpu.async_copy(m_hbm, mv, semm0).wait()

    def _fire_meta(q, par):
        pltpu.async_copy(src_hbm.at[wid, q], sv[par], semm[par])
        pltpu.async_copy(dst_hbm.at[wid, q], dv[par], semm[par])
        pltpu.async_copy(ae_hbm.at[wid, q], ab[par], semm[par])

    def _drain_meta(par):
        pltpu.make_async_copy(src_hbm.at[wid, 0], sv[par], semm[par]).wait()
        pltpu.make_async_copy(dst_hbm.at[wid, 0], dv[par], semm[par]).wait()
        pltpu.make_async_copy(ae_hbm.at[wid, 0], ab[par], semm[par]).wait()

    _fire_meta(0, 0)
    _fire_meta(1, 1)

    def _pair(g, _):
        q0 = g * 2
        for par in range(2):
            _drain_meta(par)
            pltpu.async_copy(as_hbm.at[sv[par]], gs[par], semg[par])
            pltpu.async_copy(ad_hbm.at[dv[par]], gd[par], semg[par])
        for par in range(2):
            q = q0 + par
            gbase = wid * (_NQ * 32) + q * 32
            gsp, gdp, abp = gs[par], gd[par], ab[par]
            payp, pwp = pay[par], pw[par]
            pltpu.make_async_copy(as_hbm.at[pl.ds(0, 32)], gsp,
                                  semg[par]).wait()
            pltpu.make_async_copy(as_hbm.at[pl.ds(0, 32)], gdp,
                                  semg[par]).wait()

            def _row(r, _c):
                m16 = mv[pl.ds(0, 16)]
                iota = lax.iota(jnp.int32, 16)
                ae16 = abp[r, pl.ds(0, 16)]
                al = gsp[r, pl.ds(0, 16)] + gdp[r, pl.ds(0, 16)] + ae16
                al = jnp.where(al >= 0, al, NEG_SLOPE * al)
                p16 = jnp.exp(al - m16)
                vsel = jnp.where(
                    lax.broadcast_in_dim(gbase + r, (16,), ()) < E, 1.0, 0.0)
                p16 = p16 * vsel * jnp.where(iota < H, 1.0, 0.0)
                pwp[r, pl.ds(0, 16)] = p16
                aesh = ae16 * jnp.where(
                    (iota >= H) & (iota < 2 * H), 1.0, 0.0)
                dege = vsel * jnp.where(iota == 8, 1.0, 0.0)
                payp[r, pl.ds(0, 16)] = p16 + aesh + dege
                return 0
            lax.fori_loop(0, 32, _row, 0)
            pltpu.sync_copy(payp, acc.at[dv[par]], add=True)
            pltpu.sync_copy(pwp, p_hbm.at[wid, q])
            _fire_meta(q0 + 2 + par, par)
        return 0

    lax.fori_loop(0, _NQ // 2, _pair, 0)
    _drain_meta(0)
    _drain_meta(1)

    plsc.subcore_barrier()
    pltpu.sync_copy(acc.at[pl.ds(sid * _NPT, _NPT)],
                    out_hbm.at[cid, pl.ds(sid * _NPT, _NPT)])


def _sc_attention(src2, dst2, ae16r, a16s, a16d, m16):
    mesh = plsc.VectorSubcoreMesh(core_axis_name="c", subcore_axis_name="s")
    kern = pl.kernel(
        _att_body,
        mesh=mesh,
        out_type=[
            jax.ShapeDtypeStruct((_NW, _NQ + 2, 32, 16), jnp.float32),
            jax.ShapeDtypeStruct((2, _NPAD, D_OUT), jnp.float32),
        ],
        scratch_types=(
            [pltpu.VMEM((32,), jnp.int32)] * 4
            + [pltpu.VMEM((32, 128), jnp.float32)] * 4
            + [pltpu.VMEM((32, 16), jnp.float32)] * 2
            + [pltpu.VMEM((32, D_OUT), jnp.float32)] * 2
            + [pltpu.VMEM((32, 16), jnp.float32)] * 2
            + [pltpu.VMEM((16,), jnp.float32)]
            + [pltpu.VMEM_SHARED((_NPAD, D_OUT), jnp.float32)]
            + [pltpu.SemaphoreType.DMA] * 4
        ),
    )
    return kern(src2, dst2, ae16r, a16s, a16d, m16)


_GM = 32
_NBM = 320
_NPADM = 10112
_NPTM = _NPADM // 16


def _msg_body(meta_hbm, dst_hbm, p_hbm, winv_hbm, xp4_hbm, out_hbm,
              meta0, meta1, dv0, dv1, pc0, pc1,
              gv0, gv1, ra0, rb0, rc0, rd0, ra1, rb1, rc1, rd1,
              acc, semm0, semm1, semg0, semg1):
    cid = lax.axis_index("c")
    sid = lax.axis_index("s")
    wid = sid * 2 + cid
    meta = (meta0, meta1)
    dv = (dv0, dv1)
    pc = (pc0, pc1)
    gv = (gv0, gv1)
    rows = ((ra0, rb0, rc0, rd0), (ra1, rb1, rc1, rd1))
    semm = (semm0, semm1)
    semg = (semg0, semg1)

    def _zrow(i, _):
        for v in range(8):
            rd0[i, pl.ds(v * 16, 16)] = jnp.zeros((16,), jnp.float32)
        return 0
    lax.fori_loop(0, _GM, _zrow, 0)
    for z in range(19):
        pltpu.sync_copy(rd0, acc.at[pl.ds(sid * _NPTM + z * _GM, _GM)])
    pltpu.sync_copy(rd0.at[pl.ds(0, 24)],
                    acc.at[pl.ds(sid * _NPTM + 608, 24)])
    plsc.subcore_barrier()

    def _fire_meta(b, par):
        pltpu.async_copy(meta_hbm.at[wid, b], meta[par], semm[par])
        pltpu.async_copy(dst_hbm.at[wid, b], dv[par], semm[par])
        pltpu.async_copy(p_hbm.at[wid, b], pc[par], semm[par])

    def _drain_meta(par):
        pltpu.make_async_copy(meta_hbm.at[wid, 0], meta[par],
                              semm[par]).wait()
        pltpu.make_async_copy(dst_hbm.at[wid, 0], dv[par], semm[par]).wait()
        pltpu.make_async_copy(p_hbm.at[wid, 0], pc[par], semm[par]).wait()

    def _fire_gathers(par):
        for h in range(H):
            pltpu.async_copy(
                xp4_hbm.at[h].at[meta[par]], rows[par][h], semg[par])
        pltpu.async_copy(winv_hbm.at[dv[par]], gv[par], semg[par])

    def _drain_gathers(par):
        for h in range(H):
            pltpu.make_async_copy(winv_hbm.at[pl.ds(0, _GM)],
                                  rows[par][h], semg[par]).wait()
        pltpu.make_async_copy(winv_hbm.at[pl.ds(0, _GM)], gv[par],
                              semg[par]).wait()

    _fire_meta(0, 0)
    _fire_meta(1, 1)

    def _pair(g, _):
        b0 = g * 2
        for par in range(2):
            _drain_meta(par)
            _fire_gathers(par)
        for par in range(2):
            r0p, r1p, r2p, r3p = rows[par]
            msg = r3p
            pcp = pc[par]
            gvp = gv[par]
            _drain_gathers(par)

            def _row(r, _c):
                wrow = (pcp[r // 8, pl.ds((r % 8) * 16, 16)]
                        * gvp[r, pl.ds(0, 16)])
                w0 = lax.broadcast_in_dim(wrow[0], (16,), ())
                w1 = lax.broadcast_in_dim(wrow[1], (16,), ())
                w2 = lax.broadcast_in_dim(wrow[2], (16,), ())
                w3 = lax.broadcast_in_dim(wrow[3], (16,), ())
                for v in range(8):
                    sl = pl.ds(v * 16, 16)
                    a = r0p[r, sl] * w0 + r1p[r, sl] * w1
                    a = a + r2p[r, sl] * w2 + r3p[r, sl] * w3
                    msg[r, sl] = a
                return 0
            lax.fori_loop(0, _GM, _row, 0)
            pltpu.sync_copy(msg, acc.at[dv[par]], add=True)
            _fire_meta(b0 + 2 + par, par)
        return 0

    lax.fori_loop(0, _NBM // 2, _pair, 0)
    _drain_meta(0)
    _drain_meta(1)

    plsc.subcore_barrier()
    pltpu.sync_copy(acc.at[pl.ds(sid * _NPTM, _NPTM)],
                    out_hbm.at[cid, pl.ds(sid * _NPTM, _NPTM)])


def _sc_message(meta, dst3, p3, winvtbl, xp4):
    mesh = plsc.VectorSubcoreMesh(core_axis_name="c", subcore_axis_name="s")
    kern = pl.kernel(
        _msg_body,
        mesh=mesh,
        out_type=jax.ShapeDtypeStruct((2, _NPADM, D_OUT), jnp.float32),
        scratch_types=(
            [pltpu.VMEM((_GM,), jnp.int32)] * 2
            + [pltpu.VMEM((_GM,), jnp.int32)] * 2
            + [pltpu.VMEM((4, 128), jnp.float32)] * 2
            + [pltpu.VMEM((_GM, D_OUT), jnp.float32)] * 2
            + [pltpu.VMEM((_GM, D_OUT), jnp.float32)] * 8
            + [pltpu.VMEM_SHARED((_NPADM, D_OUT), jnp.float32)]
            + [pltpu.SemaphoreType.DMA] * 4
        ),
    )
    return kern(meta, dst3, p3, winvtbl, xp4)


def kernel(x, edge_index, edge_attr, W, att_src, att_dst, W_e, att_edge):
    src, dst = edge_index[0], edge_index[1]
    B = jnp.einsum('khd,hd->kh', W_e.reshape(D_EDGE, H, D_OUT), att_edge)

    attbig = jnp.zeros((H * D_OUT, 256), jnp.float32)
    attbig = attbig.at[:, :H].set(
        jax.scipy.linalg.block_diag(*[att_src[h][:, None] for h in range(H)]))
    attbig = attbig.at[:, 128:128 + H].set(
        jax.scipy.linalg.block_diag(*[att_dst[h][:, None] for h in range(H)]))
    xpad = jnp.zeros((_NPAD, D_IN), jnp.float32).at[:N].set(x)
    xp4, a16s, a16d, bm = _project(xpad, W, attbig)
    asrc = a16s[:N, :H]
    adst = a16d[:N, :H]
    msd = jnp.max(bm.reshape(_NBLK, 256).T, axis=1)

    ae = edge_attr @ B
    ae16r = jnp.zeros((_EPAD, 16), jnp.float32)
    ae16r = ae16r.at[:E, :H].set(ae).at[:E, H:2 * H].set(ae)
    ae16r = jnp.zeros((_NW, _NQ + 2, 32, 16), jnp.float32).at[:, :_NQ].set(
        ae16r.reshape(_NW, _NQ, 32, 16))
    aemax = jnp.max(ae.T, axis=1)
    M = _lr(msd[:H] + msd[128:128 + H] + jnp.maximum(aemax, 0.0))
    m16 = jnp.zeros((16,), jnp.float32).at[:H].set(M)

    srcp = jnp.zeros((_EPAD,), jnp.int32).at[:E].set(src)
    dstp = jnp.zeros((_EPAD,), jnp.int32).at[:E].set(dst)
    src2 = jnp.zeros((_NW, _NQ + 2, 32), jnp.int32).at[:, :_NQ].set(
        srcp.reshape(_NW, _NQ, 32))
    dst2 = jnp.zeros((_NW, _NQ + 2, 32), jnp.int32).at[:, :_NQ].set(
        dstp.reshape(_NW, _NQ, 32))
    prq, sparts = _sc_attention(src2, dst2, ae16r, a16s, a16d, m16)

    sacc = sparts[0, :N] + sparts[1, :N]
    deg = sacc[:, 8]
    sA = sacc[:, 4:4 + H]
    sE = sacc[:, :H]
    a_loop = _lr(asrc + adst + sA / jnp.maximum(deg, 1.0)[:, None])
    p_loop = jnp.exp(a_loop - M[None, :])
    s = sE + p_loop
    winv = 1.0 / s
    w_loop = p_loop * winv

    winvtbl = jnp.zeros((_NPAD, 128), jnp.float32).at[:N, :H].set(winv)
    p3 = prq.reshape(_NW, _NBM + 2, 4, 128)
    parts = _sc_message(src2, dst2, p3, winvtbl, xp4)

    self_msg = jnp.zeros((N, D_OUT), jnp.float32)
    for h in range(H):
        self_msg = self_msg + w_loop[:, h:h + 1] * xp4[h, :N]
    out = (parts[0, :N] + parts[1, :N] + self_msg) * (1.0 / H)
    return out


# speedup vs baseline: 1.0705x; 1.0209x over previous
import functools

import jax
import jax.numpy as jnp
from jax import lax
from jax.experimental import pallas as pl
from jax.experimental.pallas import tpu as pltpu
from jax.experimental.pallas import tpu_sc as plsc

N = 10000
E = 320000
D_IN = 128
D_OUT = 128
H = 4
D_EDGE = 16
NEG_SLOPE = 0.2

_NPAD = 10240
_NBLK = 16

_NW = 32
_G = 128
_NB = 80
_EPAD = _NW * _NB * _G
_NPT = _NPAD // 16


def _lr(v):
    return jnp.where(v >= 0, v, NEG_SLOPE * v)


def _proj_body(x_ref, w_ref, ab_ref, xp_ref, as_ref, ad_ref, bm_ref):
    x = x_ref[...]
    xp = jnp.dot(x, w_ref[...], preferred_element_type=jnp.float32)
    a256 = jnp.dot(xp, ab_ref[...], preferred_element_type=jnp.float32)
    as_ref[...] = a256[:, :128]
    ad_ref[...] = a256[:, 128:]
    bm_ref[...] = jnp.max(a256, axis=0, keepdims=True)[None]
    for h in range(H):
        xp_ref[h] = xp[:, h * D_OUT:(h + 1) * D_OUT]


def _project(xpad, W, attbig):
    blk = _NPAD // _NBLK
    return pl.pallas_call(
        _proj_body,
        grid=(_NBLK,),
        in_specs=[
            pl.BlockSpec((blk, D_IN), lambda i: (i, 0)),
            pl.BlockSpec((D_IN, H * D_OUT), lambda i: (0, 0)),
            pl.BlockSpec((H * D_OUT, 256), lambda i: (0, 0)),
        ],
        out_specs=[
            pl.BlockSpec((H, blk, D_OUT), lambda i: (0, i, 0)),
            pl.BlockSpec((blk, 128), lambda i: (i, 0)),
            pl.BlockSpec((blk, 128), lambda i: (i, 0)),
            pl.BlockSpec((1, 1, 256), lambda i: (i, 0, 0)),
        ],
        out_shape=[
            jax.ShapeDtypeStruct((H, _NPAD, D_OUT), jnp.float32),
            jax.ShapeDtypeStruct((_NPAD, 128), jnp.float32),
            jax.ShapeDtypeStruct((_NPAD, 128), jnp.float32),
            jax.ShapeDtypeStruct((_NBLK, 1, 256), jnp.float32),
        ],
    )(xpad, W, attbig)


_NQ = 320


def _att_body(src_hbm, dst_hbm, ae_hbm, as_hbm, ad_hbm, m_hbm,
              p_hbm, out_hbm,
              sv0, sv1, dv0, dv1, gs0, gs1, gd0, gd1, ab0, ab1,
              pay0, pay1, pw0, pw1, mv, acc, semm0, semm1, semg0, semg1):
    cid = lax.axis_index("c")
    sid = lax.axis_index("s")
    wid = sid * 2 + cid
    sv = (sv0, sv1)
    dv = (dv0, dv1)
    gs = (gs0, gs1)
    gd = (gd0, gd1)
    ab = (ab0, ab1)
    pay = (pay0, pay1)
    pw = (pw0, pw1)
    semm = (semm0, semm1)
    semg = (semg0, semg1)

    def _zrow(i, _):
        for v in range(8):
            pay0[i, pl.ds(v * 16, 16)] = jnp.zeros((16,), jnp.float32)
        return 0
    lax.fori_loop(0, 32, _zrow, 0)
    for z in range(_NPT // 32):
        pltpu.sync_copy(pay0, acc.at[pl.ds(sid * _NPT + z * 32, 32)])
    plsc.subcore_barrier()

    pltpu.async_copy(m_hbm, mv, semm0).wait()

    def _fire_meta(q, par):
        pltpu.async_copy(src_hbm.at[wid, q], sv[par], semm[par])
        pltpu.async_copy(dst_hbm.at[wid, q], dv[par], semm[par])
        pltpu.async_copy(ae_hbm.at[wid, q], ab[par], semm[par])

    def _drain_meta(par):
        pltpu.make_async_copy(src_hbm.at[wid, 0], sv[par], semm[par]).wait()
        pltpu.make_async_copy(dst_hbm.at[wid, 0], dv[par], semm[par]).wait()
        pltpu.make_async_copy(ae_hbm.at[wid, 0], ab[par], semm[par]).wait()

    _fire_meta(0, 0)
    _fire_meta(1, 1)

    def _pair(g, _):
        q0 = g * 2
        for par in range(2):
            _drain_meta(par)
            pltpu.async_copy(as_hbm.at[sv[par]], gs[par], semg[par])
            pltpu.async_copy(ad_hbm.at[dv[par]], gd[par], semg[par])
        for par in range(2):
            q = q0 + par
            gbase = wid * (_NQ * 32) + q * 32
            gsp, gdp, abp = gs[par], gd[par], ab[par]
            payp, pwp = pay[par], pw[par]
            pltpu.make_async_copy(as_hbm.at[pl.ds(0, 32)], gsp,
                                  semg[par]).wait()
            pltpu.make_async_copy(as_hbm.at[pl.ds(0, 32)], gdp,
                                  semg[par]).wait()

            def _row(r, _c):
                m16 = mv[pl.ds(0, 16)]
                iota = lax.iota(jnp.int32, 16)
                ae16 = abp[r, pl.ds(0, 16)]
                al = gsp[r, pl.ds(0, 16)] + gdp[r, pl.ds(0, 16)] + ae16
                al = jnp.where(al >= 0, al, NEG_SLOPE * al)
                p16 = jnp.exp(al - m16)
                vsel = jnp.where(
                    lax.broadcast_in_dim(gbase + r, (16,), ()) < E, 1.0, 0.0)
                p16 = p16 * vsel * jnp.where(iota < H, 1.0, 0.0)
                pwp[r, pl.ds(0, 16)] = p16
                aesh = ae16 * jnp.where(
                    (iota >= H) & (iota < 2 * H), 1.0, 0.0)
                dege = vsel * jnp.where(iota == 8, 1.0, 0.0)
                payp[r, pl.ds(0, 16)] = p16 + aesh + dege
                return 0
            lax.fori_loop(0, 32, _row, 0)
            pltpu.sync_copy(payp, acc.at[dv[par]], add=True)
            pltpu.sync_copy(pwp, p_hbm.at[wid, q])
            _fire_meta(q0 + 2 + par, par)
        return 0

    lax.fori_loop(0, _NQ // 2, _pair, 0)
    _drain_meta(0)
    _drain_meta(1)

    plsc.subcore_barrier()
    pltpu.sync_copy(acc.at[pl.ds(sid * _NPT, _NPT)],
                    out_hbm.at[cid, pl.ds(sid * _NPT, _NPT)])


def _sc_attention(src2, dst2, ae16r, a16s, a16d, m16):
    mesh = plsc.VectorSubcoreMesh(core_axis_name="c", subcore_axis_name="s")
    kern = pl.kernel(
        _att_body,
        mesh=mesh,
        out_type=[
            jax.ShapeDtypeStruct((_NW, _NQ + 2, 32, 16), jnp.float32),
            jax.ShapeDtypeStruct((2, _NPAD, D_OUT), jnp.float32),
        ],
        scratch_types=(
            [pltpu.VMEM((32,), jnp.int32)] * 4
            + [pltpu.VMEM((32, 128), jnp.float32)] * 4
            + [pltpu.VMEM((32, 16), jnp.float32)] * 2
            + [pltpu.VMEM((32, D_OUT), jnp.float32)] * 2
            + [pltpu.VMEM((32, 16), jnp.float32)] * 2
            + [pltpu.VMEM((16,), jnp.float32)]
            + [pltpu.VMEM_SHARED((_NPAD, D_OUT), jnp.float32)]
            + [pltpu.SemaphoreType.DMA] * 4
        ),
    )
    return kern(src2, dst2, ae16r, a16s, a16d, m16)


_GM = 32
_NBM = 320
_NPADM = 10112
_NPTM = _NPADM // 16


def _msg_body(meta_hbm, dst_hbm, p_hbm, winv_hbm, xp4_hbm, out_hbm,
              meta0, meta1, dv0, dv1, ds0, ds1, pc0, pc1,
              gv0, gv1, ra0, rb0, rc0, rd0, ra1, rb1, rc1, rd1,
              acc, semm0, semm1, semg0, semg1, sems0, sems1):
    cid = lax.axis_index("c")
    sid = lax.axis_index("s")
    wid = sid * 2 + cid
    meta = (meta0, meta1)
    dv = (dv0, dv1)
    dvs = (ds0, ds1)
    pc = (pc0, pc1)
    gv = (gv0, gv1)
    rows = ((ra0, rb0, rc0, rd0), (ra1, rb1, rc1, rd1))
    semm = (semm0, semm1)
    semg = (semg0, semg1)
    sems = (sems0, sems1)

    def _zrow(i, _):
        for v in range(8):
            rd0[i, pl.ds(v * 16, 16)] = jnp.zeros((16,), jnp.float32)
        return 0
    lax.fori_loop(0, _GM, _zrow, 0)
    for z in range(19):
        pltpu.sync_copy(rd0, acc.at[pl.ds(sid * _NPTM + z * _GM, _GM)])
    pltpu.sync_copy(rd0.at[pl.ds(0, 24)],
                    acc.at[pl.ds(sid * _NPTM + 608, 24)])
    plsc.subcore_barrier()

    def _fire_meta(b, par):
        pltpu.async_copy(meta_hbm.at[wid, b], meta[par], semm[par])
        pltpu.async_copy(dst_hbm.at[wid, b], dv[par], semm[par])
        pltpu.async_copy(p_hbm.at[wid, b], pc[par], semm[par])

    def _drain_meta(par):
        pltpu.make_async_copy(meta_hbm.at[wid, 0], meta[par],
                              semm[par]).wait()
        pltpu.make_async_copy(dst_hbm.at[wid, 0], dv[par], semm[par]).wait()
        pltpu.make_async_copy(p_hbm.at[wid, 0], pc[par], semm[par]).wait()

    def _fire_gathers(par):
        for h in range(H):
            pltpu.async_copy(
                xp4_hbm.at[h].at[meta[par]], rows[par][h], semg[par])
        pltpu.async_copy(winv_hbm.at[dv[par]], gv[par], semg[par])

    def _drain_gathers(par):
        for h in range(H):
            pltpu.make_async_copy(winv_hbm.at[pl.ds(0, _GM)],
                                  rows[par][h], semg[par]).wait()
        pltpu.make_async_copy(winv_hbm.at[pl.ds(0, _GM)], gv[par],
                              semg[par]).wait()
        pltpu.make_async_copy(dst_hbm.at[wid, 0], dvs[par],
                              semg[par]).wait()

    _fire_meta(0, 0)
    _fire_meta(1, 1)
    for par in range(2):
        pltpu.async_copy(winv_hbm.at[pl.ds(0, _GM)], rows[par][3], sems[par])

    def _pair(g, _):
        b0 = g * 2
        for par in range(2):
            pltpu.make_async_copy(winv_hbm.at[pl.ds(0, _GM)],
                                  rows[par][3], sems[par]).wait()
            _drain_meta(par)
            pltpu.async_copy(dst_hbm.at[wid, b0 + par], dvs[par], semg[par])
            _fire_gathers(par)
        for par in range(2):
            r0p, r1p, r2p, r3p = rows[par]
            msg = r3p
            pcp = pc[par]
            gvp = gv[par]
            _drain_gathers(par)

            def _row(r, _c):
                wrow = (pcp[r // 8, pl.ds((r % 8) * 16, 16)]
                        * gvp[r, pl.ds(0, 16)])
                w0 = lax.broadcast_in_dim(wrow[0], (16,), ())
                w1 = lax.broadcast_in_dim(wrow[1], (16,), ())
                w2 = lax.broadcast_in_dim(wrow[2], (16,), ())
                w3 = lax.broadcast_in_dim(wrow[3], (16,), ())
                for v in range(8):
                    sl = pl.ds(v * 16, 16)
                    a = r0p[r, sl] * w0 + r1p[r, sl] * w1
                    a = a + r2p[r, sl] * w2 + r3p[r, sl] * w3
                    msg[r, sl] = a
                return 0
            lax.fori_loop(0, _GM, _row, 0)
            pltpu.async_copy(msg, acc.at[dvs[par]], sems[par], add=True)
            _fire_meta(b0 + 2 + par, par)
        return 0

    lax.fori_loop(0, _NBM // 2, _pair, 0)
    _drain_meta(0)
    _drain_meta(1)
    for par in range(2):
        pltpu.make_async_copy(winv_hbm.at[pl.ds(0, _GM)],
                              rows[par][3], sems[par]).wait()

    plsc.subcore_barrier()
    pltpu.sync_copy(acc.at[pl.ds(sid * _NPTM, _NPTM)],
                    out_hbm.at[cid, pl.ds(sid * _NPTM, _NPTM)])


def _sc_message(meta, dst3, p3, winvtbl, xp4):
    mesh = plsc.VectorSubcoreMesh(core_axis_name="c", subcore_axis_name="s")
    kern = pl.kernel(
        _msg_body,
        mesh=mesh,
        out_type=jax.ShapeDtypeStruct((2, _NPADM, D_OUT), jnp.float32),
        scratch_types=(
            [pltpu.VMEM((_GM,), jnp.int32)] * 2
            + [pltpu.VMEM((_GM,), jnp.int32)] * 4
            + [pltpu.VMEM((4, 128), jnp.float32)] * 2
            + [pltpu.VMEM((_GM, D_OUT), jnp.float32)] * 2
            + [pltpu.VMEM((_GM, D_OUT), jnp.float32)] * 8
            + [pltpu.VMEM_SHARED((_NPADM, D_OUT), jnp.float32)]
            + [pltpu.SemaphoreType.DMA] * 6
        ),
    )
    return kern(meta, dst3, p3, winvtbl, xp4)


def kernel(x, edge_index, edge_attr, W, att_src, att_dst, W_e, att_edge):
    src, dst = edge_index[0], edge_index[1]
    B = jnp.einsum('khd,hd->kh', W_e.reshape(D_EDGE, H, D_OUT), att_edge)

    attbig = jnp.zeros((H * D_OUT, 256), jnp.float32)
    attbig = attbig.at[:, :H].set(
        jax.scipy.linalg.block_diag(*[att_src[h][:, None] for h in range(H)]))
    attbig = attbig.at[:, 128:128 + H].set(
        jax.scipy.linalg.block_diag(*[att_dst[h][:, None] for h in range(H)]))
    xpad = jnp.zeros((_NPAD, D_IN), jnp.float32).at[:N].set(x)
    xp4, a16s, a16d, bm = _project(xpad, W, attbig)
    asrc = a16s[:N, :H]
    adst = a16d[:N, :H]
    msd = jnp.max(bm.reshape(_NBLK, 256).T, axis=1)

    ae = edge_attr @ B
    ae16r = jnp.zeros((_EPAD, 16), jnp.float32)
    ae16r = ae16r.at[:E, :H].set(ae).at[:E, H:2 * H].set(ae)
    ae16r = jnp.zeros((_NW, _NQ + 2, 32, 16), jnp.float32).at[:, :_NQ].set(
        ae16r.reshape(_NW, _NQ, 32, 16))
    aemax = jnp.max(ae.T, axis=1)
    M = _lr(msd[:H] + msd[128:128 + H] + jnp.maximum(aemax, 0.0))
    m16 = jnp.zeros((16,), jnp.float32).at[:H].set(M)

    srcp = jnp.zeros((_EPAD,), jnp.int32).at[:E].set(src)
    dstp = jnp.zeros((_EPAD,), jnp.int32).at[:E].set(dst)
    src2 = jnp.zeros((_NW, _NQ + 2, 32), jnp.int32).at[:, :_NQ].set(
        srcp.reshape(_NW, _NQ, 32))
    dst2 = jnp.zeros((_NW, _NQ + 2, 32), jnp.int32).at[:, :_NQ].set(
        dstp.reshape(_NW, _NQ, 32))
    prq, sparts = _sc_attention(src2, dst2, ae16r, a16s, a16d, m16)

    sacc = sparts[0, :N] + sparts[1, :N]
    deg = sacc[:, 8]
    sA = sacc[:, 4:4 + H]
    sE = sacc[:, :H]
    a_loop = _lr(asrc + adst + sA / jnp.maximum(deg, 1.0)[:, None])
    p_loop = jnp.exp(a_loop - M[None, :])
    s = sE + p_loop
    winv = 1.0 / s
    w_loop = p_loop * winv

    winvtbl = jnp.zeros((_NPAD, 128), jnp.float32).at[:N, :H].set(winv)
    p3 = prq.reshape(_NW, _NBM + 2, 4, 128)
    parts = _sc_message(src2, dst2, p3, winvtbl, xp4)

    self_msg = jnp.zeros((N, D_OUT), jnp.float32)
    for h in range(H):
        self_msg = self_msg + w_loop[:, h:h + 1] * xp4[h, :N]
    out = (parts[0, :N] + parts[1, :N] + self_msg) * (1.0 / H)
    return out
